# Initial kernel scaffold; baseline (speedup 1.0000x reference)
import jax
import jax.numpy as jnp
from jax.experimental import pallas as pl


def kernel(all_tubes, gt_boxes, gt_tubes_all):
    raise NotImplementedError("write your pallas kernel here")


# speedup vs baseline: 1.3197x; 1.3197x over previous
import functools

import jax
import jax.numpy as jnp
from jax.experimental import pallas as pl

T = 16
B = 4
N_TUBES = 5000
N_ACTIONS = 8
R = N_TUBES + N_ACTIONS
ROIS = 128
FG_PER_IMAGE = 32
FG_THRESH = 0.5
BG_THRESH_HI = 0.5
BG_THRESH_LO = 0.1

_HIGHEST = jax.lax.Precision.HIGHEST


def _dot(a, b):
    return jnp.dot(a, b, precision=_HIGHEST, preferred_element_type=jnp.float32)


def _fiota(shape, dim):
    return jax.lax.broadcasted_iota(jnp.int32, shape, dim).astype(jnp.float32)


def _excl_scan(m):
    n = m.shape[0]
    x = m
    s = 1
    while s < n:
        shifted = jnp.concatenate([jnp.zeros((s, 1), jnp.float32), x[: n - s]], axis=0)
        x = x + shifted
        s *= 2
    return x - m


def _interleave(dx, dy, dw, dh):
    r = _fiota((T, 4 * T), 0)
    c = _fiota((T, 4 * T), 1)
    out = _dot(dx, (c == 4.0 * r).astype(jnp.float32))
    out = out + _dot(dy, (c == 4.0 * r + 1.0).astype(jnp.float32))
    out = out + _dot(dw, (c == 4.0 * r + 2.0).astype(jnp.float32))
    out = out + _dot(dh, (c == 4.0 * r + 3.0).astype(jnp.float32))
    return out


def _kernel(tubes_ref, gx1_ref, gy1_ref, gx2_ref, gy2_ref, glab_ref,
            rois_ref, tubes_out_ref, lab_ref, bt_ref, bi_ref, bo_ref):
    b = pl.program_id(0)
    tf = tubes_ref[0]
    gx1 = gx1_ref[0]
    gy1 = gy1_ref[0]
    gx2 = gx2_ref[0]
    gy2 = gy2_ref[0]
    glab = glab_ref[0]

    bx1 = tf[:N_TUBES, 1:2]
    by1 = tf[:N_TUBES, 2:3]
    st = jnp.round(tf[:N_TUBES, 3:4])
    bx2 = tf[:N_TUBES, 4:5]
    by2 = tf[:N_TUBES, 5:6]
    en = jnp.round(tf[:N_TUBES, 6:7])
    fr = _fiota((N_TUBES, T), 1)
    mask = ((fr >= st) & (fr <= en)).astype(jnp.float32)
    rx1 = jnp.concatenate([mask * bx1, gx1], axis=0)
    ry1 = jnp.concatenate([mask * by1, gy1], axis=0)
    rx2 = jnp.concatenate([mask * bx2, gx2], axis=0)
    ry2 = jnp.concatenate([mask * by2, gy2], axis=0)

    r_empty = (jnp.abs(rx1) + jnp.abs(ry1) + jnp.abs(rx2) + jnp.abs(ry2)) == 0.0
    g_empty = (jnp.abs(gx1) + jnp.abs(gy1) + jnp.abs(gx2) + jnp.abs(gy2)) == 0.0
    ra = (rx2 - rx1 + 1.0) * (ry2 - ry1 + 1.0)
    ga = (gx2 - gx1 + 1.0) * (gy2 - gy1 + 1.0)

    ov_cols = []
    for j in range(N_ACTIONS):
        jx1 = gx1[j : j + 1]
        jy1 = gy1[j : j + 1]
        jx2 = gx2[j : j + 1]
        jy2 = gy2[j : j + 1]
        iw = jnp.clip(jnp.minimum(rx2, jx2) - jnp.maximum(rx1, jx1) + 1.0, 0.0)
        ih = jnp.clip(jnp.minimum(ry2, jy2) - jnp.maximum(ry1, jy1) + 1.0, 0.0)
        inter = iw * ih
        union = jnp.maximum(ra + ga[j : j + 1] - inter, 1e-6)
        either_empty = r_empty | g_empty[j : j + 1]
        iou = jnp.where(either_empty, 0.0, inter / union)
        denom = jnp.maximum(
            jnp.sum((~(r_empty & g_empty[j : j + 1])).astype(jnp.float32),
                    axis=1, keepdims=True), 1.0)
        ov_cols.append(jnp.sum(iou, axis=1, keepdims=True) / denom)
    ov = jnp.concatenate(ov_cols, axis=1)

    mx = jnp.max(ov, axis=1, keepdims=True)
    j8 = _fiota((R, N_ACTIONS), 1)
    asn = jnp.min(jnp.where(ov >= mx, j8, 99.0), axis=1, keepdims=True)

    has = glab != 0.0
    has_any = jnp.any(has, axis=1, keepdims=True)
    ft = _fiota((N_ACTIONS, T), 1)
    first = jnp.min(jnp.where(has, ft, 99.0), axis=1, keepdims=True)
    lab0 = jnp.sum(jnp.where(ft == first, glab, 0.0), axis=1, keepdims=True)
    lab0 = jnp.where(has_any, lab0, 0.0)

    fgm = (mx >= FG_THRESH).astype(jnp.float32)
    bgm = ((mx < BG_THRESH_HI) & (mx >= BG_THRESH_LO)).astype(jnp.float32)
    fgn = jnp.sum(fgm, keepdims=True)
    bgn = jnp.sum(bgm, keepdims=True)
    fg_rank = _excl_scan(fgm)
    bg_rank = _excl_scan(bgm)
    k128 = _fiota((R, ROIS), 1)
    idxc = _fiota((R, ROIS), 0)
    fg_order = jnp.sum(
        jnp.where((fgm > 0.0) & (fg_rank == k128), idxc, 0.0), axis=0, keepdims=True)
    bg_order = jnp.sum(
        jnp.where((bgm > 0.0) & (bg_rank == k128), idxc, 0.0), axis=0, keepdims=True)

    arc = _fiota((ROIS, 1), 0)
    both = (fgn > 0.0) & (bgn > 0.0)
    fg_only = (fgn > 0.0) & (bgn == 0.0)
    bg_only = (fgn == 0.0) & (bgn > 0.0)
    fg_this = jnp.where(both, jnp.minimum(float(FG_PER_IMAGE), fgn),
                        jnp.where(fg_only, float(ROIS), 0.0))

    lane128 = _fiota((ROIS, ROIS), 1)

    def gather128(table, pos):
        return jnp.sum(jnp.where(pos == lane128, table, 0.0), axis=1, keepdims=True)

    def fmod(a, n):
        return a - jnp.floor(a / n) * n

    keep_both = jnp.where(
        arc < fg_this,
        gather128(fg_order, jnp.minimum(arc, jnp.maximum(fgn - 1.0, 0.0))),
        gather128(bg_order, fmod(jnp.maximum(arc - fg_this, 0.0),
                                 jnp.maximum(bgn, 1.0))))
    keep_fgo = gather128(fg_order, fmod(arc, jnp.maximum(fgn, 1.0)))
    keep_bgo = gather128(bg_order, fmod(arc, jnp.maximum(bgn, 1.0)))
    keep = jnp.where(both, keep_both, jnp.where(fg_only, keep_fgo, keep_bgo))

    has_gt = jnp.max(lab0, keepdims=True) > 0.0
    valid = (both | fg_only | bg_only) & has_gt
    vf = valid.astype(jnp.float32)

    koh = (keep == _fiota((ROIS, R), 1)).astype(jnp.float32)
    a_keep = _dot(koh, asn)
    oh8 = (a_keep == _fiota((ROIS, N_ACTIONS), 1)).astype(jnp.float32)

    lab_k = _dot(oh8, lab0)
    lab_k = jnp.where(arc < fg_this, lab_k, 0.0)
    lab_k = jnp.where(valid, lab_k, 0.0)

    ex1 = _dot(koh, rx1) * vf
    ey1 = _dot(koh, ry1) * vf
    ex2 = _dot(koh, rx2) * vf
    ey2 = _dot(koh, ry2) * vf
    tub_k = _dot(koh, tf) * vf
    gx1k = _dot(oh8, gx1) * vf
    gy1k = _dot(oh8, gy1) * vf
    gx2k = _dot(oh8, gx2) * vf
    gy2k = _dot(oh8, gy2) * vf

    rois64 = _interleave(ex1, ey1, ex2, ey2)
    bcol = jnp.full((ROIS, 1), 1.0, jnp.float32) * b.astype(jnp.float32)
    rois_ref[0] = jnp.concatenate([bcol, rois64], axis=1)
    tubes_out_ref[0] = tub_k
    lab_ref[0] = lab_k

    ew = jnp.maximum(ex2 - ex1 + 1.0, 1e-3)
    eh = jnp.maximum(ey2 - ey1 + 1.0, 1e-3)
    ecx = ex1 + 0.5 * ew
    ecy = ey1 + 0.5 * eh
    gw = jnp.maximum(gx2k - gx1k + 1.0, 1e-3)
    gh = jnp.maximum(gy2k - gy1k + 1.0, 1e-3)
    gcx = gx1k + 0.5 * gw
    gcy = gy1k + 0.5 * gh
    dx = (gcx - ecx) / ew
    dy = (gcy - ecy) / eh
    dw = jnp.log(gw / ew)
    dh = jnp.log(gh / eh)
    t64 = _interleave(dx, dy, dw, dh)

    lane64 = _fiota((1, 4 * T), 1)
    m4 = lane64 - 4.0 * jnp.floor(lane64 / 4.0)
    stds = jnp.where(m4 < 2.0, 0.1, 0.2)
    t64 = t64 / stds

    pos = (lab_k > 0.0).astype(jnp.float32)
    bt_ref[0] = t64 * pos
    inside = pos * jnp.ones((1, 4 * T), jnp.float32)
    bi_ref[0] = inside
    bo_ref[0] = (inside > 0.0).astype(jnp.float32)


@jax.jit
def kernel(all_tubes, gt_boxes, gt_tubes_all):
    b = gt_boxes.shape[0]
    na = gt_boxes.shape[1]
    gt_append = jnp.concatenate(
        [jnp.zeros((b, na, 1), all_tubes.dtype), gt_tubes_all[:, :, :6]], axis=2)
    tubes_full = jnp.concatenate([all_tubes, gt_append], axis=1)
    gx1 = gt_boxes[..., 0]
    gy1 = gt_boxes[..., 1]
    gx2 = gt_boxes[..., 2]
    gy2 = gt_boxes[..., 3]
    glab = gt_boxes[..., 4]

    spec3 = lambda s2, s3: pl.BlockSpec((1, s2, s3), lambda i: (i, 0, 0))
    out = pl.pallas_call(
        _kernel,
        grid=(b,),
        in_specs=[spec3(R, 7), spec3(na, T), spec3(na, T), spec3(na, T),
                  spec3(na, T), spec3(na, T)],
        out_specs=[spec3(ROIS, 4 * T + 1), spec3(ROIS, 7), spec3(ROIS, 1),
                   spec3(ROIS, 4 * T), spec3(ROIS, 4 * T), spec3(ROIS, 4 * T)],
        out_shape=[
            jax.ShapeDtypeStruct((b, ROIS, 4 * T + 1), jnp.float32),
            jax.ShapeDtypeStruct((b, ROIS, 7), jnp.float32),
            jax.ShapeDtypeStruct((b, ROIS, 1), jnp.float32),
            jax.ShapeDtypeStruct((b, ROIS, 4 * T), jnp.float32),
            jax.ShapeDtypeStruct((b, ROIS, 4 * T), jnp.float32),
            jax.ShapeDtypeStruct((b, ROIS, 4 * T), jnp.float32),
        ],
    )(tubes_full, gx1, gy1, gx2, gy2, glab)
    rois_b, tubes_b, labels_b, bt, bi, bo = out
    return rois_b, tubes_b, labels_b.reshape(b, ROIS), bt, bi, bo


# speedup vs baseline: 1.3236x; 1.0029x over previous
import functools

import jax
import jax.numpy as jnp
from jax import lax
from jax.experimental import pallas as pl
from jax.experimental.pallas import tpu as pltpu
from jax.experimental.pallas import tpu_sc as plsc

T = 16
N_TUBES = 5000
N_ACTIONS = 8
R = N_TUBES + N_ACTIONS
ROIS = 128
FG_PER_IMAGE = 32

_HIGHEST = jax.lax.Precision.HIGHEST


def _dot(a, b):
    return jnp.dot(a, b, precision=_HIGHEST, preferred_element_type=jnp.float32)


def _fiota(shape, dim):
    return jax.lax.broadcasted_iota(jnp.int32, shape, dim).astype(jnp.float32)


def _excl_scan(m):
    n = m.shape[0]
    x = m
    s = 1
    while s < n:
        shifted = jnp.concatenate([jnp.zeros((s, 1), jnp.float32), x[: n - s]], axis=0)
        x = x + shifted
        s *= 2
    return x - m


def _interleave(dx, dy, dw, dh):
    r = _fiota((T, 4 * T), 0)
    c = _fiota((T, 4 * T), 1)
    out = _dot(dx, (c == 4.0 * r).astype(jnp.float32))
    out = out + _dot(dy, (c == 4.0 * r + 1.0).astype(jnp.float32))
    out = out + _dot(dw, (c == 4.0 * r + 2.0).astype(jnp.float32))
    out = out + _dot(dh, (c == 4.0 * r + 3.0).astype(jnp.float32))
    return out


def _kernel_a(tubes_ref, gx1_ref, gy1_ref, gx2_ref, gy2_ref, glab_ref,
              tab_ref, gtab_ref, keep_ref, akeep_ref, lab_out_ref, misc_ref):
    tf = tubes_ref[0]
    gx1 = gx1_ref[0]
    gy1 = gy1_ref[0]
    gx2 = gx2_ref[0]
    gy2 = gy2_ref[0]
    glab = glab_ref[0]

    bx1 = tf[:N_TUBES, 1:2]
    by1 = tf[:N_TUBES, 2:3]
    st = jnp.round(tf[:N_TUBES, 3:4])
    bx2 = tf[:N_TUBES, 4:5]
    by2 = tf[:N_TUBES, 5:6]
    en = jnp.round(tf[:N_TUBES, 6:7])
    fr = _fiota((N_TUBES, T), 1)
    mask = ((fr >= st) & (fr <= en)).astype(jnp.float32)
    rx1 = jnp.concatenate([mask * bx1, gx1], axis=0)
    ry1 = jnp.concatenate([mask * by1, gy1], axis=0)
    rx2 = jnp.concatenate([mask * bx2, gx2], axis=0)
    ry2 = jnp.concatenate([mask * by2, gy2], axis=0)

    tab_ref[0] = jnp.concatenate(
        [rx1, ry1, rx2, ry2, tf, jnp.zeros((R, 128 - 4 * T - 7), jnp.float32)],
        axis=1)

    r_empty = (jnp.abs(rx1) + jnp.abs(ry1) + jnp.abs(rx2) + jnp.abs(ry2)) == 0.0
    g_empty = (jnp.abs(gx1) + jnp.abs(gy1) + jnp.abs(gx2) + jnp.abs(gy2)) == 0.0
    ra = (rx2 - rx1 + 1.0) * (ry2 - ry1 + 1.0)
    ga = (gx2 - gx1 + 1.0) * (gy2 - gy1 + 1.0)

    ov_cols = []
    for j in range(N_ACTIONS):
        iw = jnp.clip(jnp.minimum(rx2, gx2[j:j + 1]) - jnp.maximum(rx1, gx1[j:j + 1]) + 1.0, 0.0)
        ih = jnp.clip(jnp.minimum(ry2, gy2[j:j + 1]) - jnp.maximum(ry1, gy1[j:j + 1]) + 1.0, 0.0)
        inter = iw * ih
        union = jnp.maximum(ra + ga[j:j + 1] - inter, 1e-6)
        either_empty = r_empty | g_empty[j:j + 1]
        iou = jnp.where(either_empty, 0.0, inter / union)
        denom = jnp.maximum(
            jnp.sum((~(r_empty & g_empty[j:j + 1])).astype(jnp.float32),
                    axis=1, keepdims=True), 1.0)
        ov_cols.append(jnp.sum(iou, axis=1, keepdims=True) / denom)
    ov = jnp.concatenate(ov_cols, axis=1)

    mx = jnp.max(ov, axis=1, keepdims=True)
    j8 = _fiota((R, N_ACTIONS), 1)
    asn = jnp.min(jnp.where(ov >= mx, j8, 99.0), axis=1, keepdims=True)

    has = glab != 0.0
    has_any = jnp.any(has, axis=1, keepdims=True)
    ft = _fiota((N_ACTIONS, T), 1)
    first = jnp.min(jnp.where(has, ft, 99.0), axis=1, keepdims=True)
    lab0 = jnp.sum(jnp.where(ft == first, glab, 0.0), axis=1, keepdims=True)
    lab0 = jnp.where(has_any, lab0, 0.0)

    gtab_ref[0] = jnp.concatenate(
        [gx1, gy1, gx2, gy2, lab0,
         jnp.zeros((N_ACTIONS, 128 - 4 * T - 1), jnp.float32)], axis=1)

    fgm = (mx >= 0.5).astype(jnp.float32)
    bgm = ((mx < 0.5) & (mx >= 0.1)).astype(jnp.float32)
    fgn = jnp.sum(fgm, keepdims=True)
    bgn = jnp.sum(bgm, keepdims=True)
    fg_rank = _excl_scan(fgm)
    bg_rank = _excl_scan(bgm)
    k128 = _fiota((R, ROIS), 1)
    idxc = _fiota((R, ROIS), 0)
    fg_order = jnp.sum(
        jnp.where((fgm > 0.0) & (fg_rank == k128), idxc, 0.0), axis=0, keepdims=True)
    bg_order = jnp.sum(
        jnp.where((bgm > 0.0) & (bg_rank == k128), idxc, 0.0), axis=0, keepdims=True)

    arc = _fiota((ROIS, 1), 0)
    both = (fgn > 0.0) & (bgn > 0.0)
    fg_only = (fgn > 0.0) & (bgn == 0.0)
    bg_only = (fgn == 0.0) & (bgn > 0.0)
    fg_this = jnp.where(both, jnp.minimum(float(FG_PER_IMAGE), fgn),
                        jnp.where(fg_only, float(ROIS), 0.0))

    lane128 = _fiota((ROIS, ROIS), 1)

    def gather128(table, pos):
        return jnp.sum(jnp.where(pos == lane128, table, 0.0), axis=1, keepdims=True)

    def fmod(a, n):
        return a - jnp.floor(a / n) * n

    keep_both = jnp.where(
        arc < fg_this,
        gather128(fg_order, jnp.minimum(arc, jnp.maximum(fgn - 1.0, 0.0))),
        gather128(bg_order, fmod(jnp.maximum(arc - fg_this, 0.0),
                                 jnp.maximum(bgn, 1.0))))
    keep_fgo = gather128(fg_order, fmod(arc, jnp.maximum(fgn, 1.0)))
    keep_bgo = gather128(bg_order, fmod(arc, jnp.maximum(bgn, 1.0)))
    keep = jnp.where(both, keep_both, jnp.where(fg_only, keep_fgo, keep_bgo))

    has_gt = jnp.max(lab0, keepdims=True) > 0.0
    valid = (both | fg_only | bg_only) & has_gt
    vf = valid.astype(jnp.float32)

    koh = (keep == _fiota((ROIS, R), 1)).astype(jnp.float32)
    a_keep = _dot(koh, asn)
    oh8 = (a_keep == _fiota((ROIS, N_ACTIONS), 1)).astype(jnp.float32)
    lab_k = _dot(oh8, lab0)
    lab_k = jnp.where(arc < fg_this, lab_k, 0.0)
    lab_k = jnp.where(valid, lab_k, 0.0)

    keep_ref[0] = keep
    akeep_ref[0] = a_keep
    lab_out_ref[0] = lab_k
    misc_ref[0] = vf * jnp.ones((1, 16), jnp.float32)


def _sc_body(tab_hbm, gtab_hbm, keep_hbm, akeep_hbm, rows_hbm, grows_hbm,
             idx_v, rows_v, grows_v, sem):
    wid = lax.axis_index("s") * 2 + lax.axis_index("c")
    nb = rows_hbm.shape[0]

    @pl.when(wid < nb)
    def _():
        pltpu.sync_copy(keep_hbm.at[wid], idx_v)
        for c in range(ROIS // 16):
            idx_v[pl.ds(c * 16, 16)] = idx_v[pl.ds(c * 16, 16)] + wid * R
        pltpu.async_copy(tab_hbm.at[idx_v], rows_v, sem).wait()
        pltpu.sync_copy(rows_v, rows_hbm.at[wid])

        pltpu.sync_copy(akeep_hbm.at[wid], idx_v)
        for c in range(ROIS // 16):
            idx_v[pl.ds(c * 16, 16)] = idx_v[pl.ds(c * 16, 16)] + wid * N_ACTIONS
        pltpu.async_copy(gtab_hbm.at[idx_v], grows_v, sem).wait()
        pltpu.sync_copy(grows_v, grows_hbm.at[wid])


def _sc_gather(tab, gtab, keep, akeep, nb):
    mesh = plsc.VectorSubcoreMesh(core_axis_name="c", subcore_axis_name="s")
    return pl.kernel(
        _sc_body,
        mesh=mesh,
        out_type=[
            jax.ShapeDtypeStruct((nb, ROIS, 128), jnp.float32),
            jax.ShapeDtypeStruct((nb, ROIS, 128), jnp.float32),
        ],
        scratch_types=[
            pltpu.VMEM((ROIS,), jnp.int32),
            pltpu.VMEM((ROIS, 128), jnp.float32),
            pltpu.VMEM((ROIS, 128), jnp.float32),
            pltpu.SemaphoreType.DMA,
        ],
    )(tab, gtab, keep, akeep)


def _kernel_c(rows_ref, grows_ref, lab_ref_in, misc_ref,
              rois_ref, tubes_out_ref, bt_ref, bi_ref, bo_ref):
    b = pl.program_id(0)
    rows = rows_ref[0]
    grows = grows_ref[0]
    lab_k = lab_ref_in[0]
    vf = misc_ref[0][:, 0:1]

    ex1 = rows[:, 0:T] * vf
    ey1 = rows[:, T:2 * T] * vf
    ex2 = rows[:, 2 * T:3 * T] * vf
    ey2 = rows[:, 3 * T:4 * T] * vf
    tub_k = rows[:, 4 * T:4 * T + 7] * vf
    gx1k = grows[:, 0:T] * vf
    gy1k = grows[:, T:2 * T] * vf
    gx2k = grows[:, 2 * T:3 * T] * vf
    gy2k = grows[:, 3 * T:4 * T] * vf

    rois64 = _interleave(ex1, ey1, ex2, ey2)
    bcol = jnp.full((ROIS, 1), 1.0, jnp.float32) * b.astype(jnp.float32)
    rois_ref[0] = jnp.concatenate([bcol, rois64], axis=1)
    tubes_out_ref[0] = tub_k

    ew = jnp.maximum(ex2 - ex1 + 1.0, 1e-3)
    eh = jnp.maximum(ey2 - ey1 + 1.0, 1e-3)
    ecx = ex1 + 0.5 * ew
    ecy = ey1 + 0.5 * eh
    gw = jnp.maximum(gx2k - gx1k + 1.0, 1e-3)
    gh = jnp.maximum(gy2k - gy1k + 1.0, 1e-3)
    gcx = gx1k + 0.5 * gw
    gcy = gy1k + 0.5 * gh
    dx = (gcx - ecx) / ew
    dy = (gcy - ecy) / eh
    dw = jnp.log(gw / ew)
    dh = jnp.log(gh / eh)
    t64 = _interleave(dx, dy, dw, dh)

    lane64 = _fiota((1, 4 * T), 1)
    m4 = lane64 - 4.0 * jnp.floor(lane64 / 4.0)
    stds = jnp.where(m4 < 2.0, 0.1, 0.2)
    t64 = t64 / stds

    pos = (lab_k > 0.0).astype(jnp.float32)
    bt_ref[0] = t64 * pos
    inside = pos * jnp.ones((1, 4 * T), jnp.float32)
    bi_ref[0] = inside
    bo_ref[0] = (inside > 0.0).astype(jnp.float32)


@jax.jit
def kernel(all_tubes, gt_boxes, gt_tubes_all):
    nb = gt_boxes.shape[0]
    na = gt_boxes.shape[1]
    gt_append = jnp.concatenate(
        [jnp.zeros((nb, na, 1), all_tubes.dtype), gt_tubes_all[:, :, :6]], axis=2)
    tubes_full = jnp.concatenate([all_tubes, gt_append], axis=1)
    gx1 = gt_boxes[..., 0]
    gy1 = gt_boxes[..., 1]
    gx2 = gt_boxes[..., 2]
    gy2 = gt_boxes[..., 3]
    glab = gt_boxes[..., 4]

    spec3 = lambda s2, s3: pl.BlockSpec((1, s2, s3), lambda i: (i, 0, 0))
    tab, gtab, keep, akeep, labels, misc = pl.pallas_call(
        _kernel_a,
        grid=(nb,),
        in_specs=[spec3(R, 7), spec3(na, T), spec3(na, T), spec3(na, T),
                  spec3(na, T), spec3(na, T)],
        out_specs=[spec3(R, 128), spec3(na, 128), spec3(ROIS, 1),
                   spec3(ROIS, 1), spec3(ROIS, 1), spec3(1, 16)],
        out_shape=[
            jax.ShapeDtypeStruct((nb, R, 128), jnp.float32),
            jax.ShapeDtypeStruct((nb, na, 128), jnp.float32),
            jax.ShapeDtypeStruct((nb, ROIS, 1), jnp.float32),
            jax.ShapeDtypeStruct((nb, ROIS, 1), jnp.float32),
            jax.ShapeDtypeStruct((nb, ROIS, 1), jnp.float32),
            jax.ShapeDtypeStruct((nb, 1, 16), jnp.float32),
        ],
    )(tubes_full, gx1, gy1, gx2, gy2, glab)

    keepi = keep.astype(jnp.int32).reshape(nb, ROIS)
    akeepi = akeep.astype(jnp.int32).reshape(nb, ROIS)
    rows, grows = _sc_gather(tab.reshape(nb * R, 128), gtab.reshape(nb * na, 128),
                             keepi, akeepi, nb)

    out = pl.pallas_call(
        _kernel_c,
        grid=(nb,),
        in_specs=[spec3(ROIS, 128), spec3(ROIS, 128), spec3(ROIS, 1),
                  spec3(1, 16)],
        out_specs=[spec3(ROIS, 4 * T + 1), spec3(ROIS, 7),
                   spec3(ROIS, 4 * T), spec3(ROIS, 4 * T), spec3(ROIS, 4 * T)],
        out_shape=[
            jax.ShapeDtypeStruct((nb, ROIS, 4 * T + 1), jnp.float32),
            jax.ShapeDtypeStruct((nb, ROIS, 7), jnp.float32),
            jax.ShapeDtypeStruct((nb, ROIS, 4 * T), jnp.float32),
            jax.ShapeDtypeStruct((nb, ROIS, 4 * T), jnp.float32),
            jax.ShapeDtypeStruct((nb, ROIS, 4 * T), jnp.float32),
        ],
    )(rows, grows, labels, misc)
    rois_b, tubes_b, bt, bi, bo = out
    return rois_b, tubes_b, labels.reshape(nb, ROIS), bt, bi, bo
